# Initial kernel scaffold; baseline (speedup 1.0000x reference)
import jax
import jax.numpy as jnp
from jax.experimental import pallas as pl


def kernel(coord, feat, offset, segment, params):
    raise NotImplementedError("write your pallas kernel here")


# speedup vs baseline: 1.0001x; 1.0001x over previous
import jax
import jax.numpy as jnp
from jax.experimental import pallas as pl

K_LIST = [16, 128, 16, 256]
SPACE_DIM = 4
PROP_DIM = 64
D_SHAPE = 32


def _bn(x, g, b):
    mu = jnp.mean(x, axis=0)
    var = jnp.var(x, axis=0)
    return g * (x - mu) / jnp.sqrt(var + 1e-5) + b


def _gravnet_agg(s, flr, k):
    n = s.shape[0]
    C = 500
    s2 = jnp.sum(s * s, axis=1)

    def f(qs):
        d = jnp.sum(qs * qs, axis=1)[:, None] - 2.0 * (qs @ s.T) + s2[None, :]
        negd, idx = jax.lax.top_k(-d, k + 1)
        idx = idx[:, 1:]
        dsq = jnp.clip(-negd[:, 1:], 0.0, None)
        w = jnp.exp(-10.0 * dsq)
        nb = flr[idx] * w[..., None]
        return jnp.concatenate([jnp.mean(nb, axis=1), jnp.max(nb, axis=1)], axis=1)

    out = jax.lax.map(f, s.reshape(n // C, C, s.shape[1]))
    return out.reshape(n, 2 * flr.shape[1])


def _block(x, p, i, k):
    xn = _bn(x, p['blk%d_bn_g' % i], p['blk%d_bn_b' % i])
    s = xn @ p['blk%d_Ws' % i].T + p['blk%d_bs' % i]
    flr = xn @ p['blk%d_Wf' % i].T + p['blk%d_bf' % i]
    agg = _gravnet_agg(s, flr, k)
    h = jnp.concatenate([xn, agg], axis=1)
    return jax.nn.elu(h @ p['blk%d_Wo' % i].T + p['blk%d_bo' % i])


def _final_proj_kernel(x_ref, w_ref, o_ref):
    o_ref[...] = jnp.dot(x_ref[...], w_ref[...],
                         preferred_element_type=jnp.float32)


def kernel(coord, feat, offset, segment, params):
    p = params
    x = _bn(feat, p['bn1_g'], p['bn1_b'])
    x = x @ p['dense1_W'].T
    allfeat = [x]
    for i, k in enumerate(K_LIST):
        out = _block(x, p, i, k)
        allfeat.append(out)
        x = jnp.concatenate(allfeat, axis=1)
    x = jnp.concatenate(allfeat, axis=-1)
    for i in range(3):
        x = jax.nn.elu(x @ p['post%d_W' % i].T + p['post%d_b' % i])
    x = _bn(x, p['bn2_g'], p['bn2_b'])
    wt = p['clust_W'].T
    return pl.pallas_call(
        _final_proj_kernel,
        out_shape=jax.ShapeDtypeStruct((x.shape[0], wt.shape[1]), jnp.float32),
    )(x, wt)


# speedup vs baseline: 1.1042x; 1.1041x over previous
import functools

import jax
import jax.numpy as jnp
from jax.experimental import pallas as pl

K_LIST = [16, 128, 16, 256]
SPACE_DIM = 4
PROP_DIM = 64
D_SHAPE = 32

N = 10000
NPAD = 10240
Q = 256
C = 500
BIG = 3.0e38
PAD_COORD = 1.0e18
INT_MIN = -2147483648


def _select_kernel_body(k, s_ref, st_ref, q2_ref, s2_ref, out_ref):
    qs = s_ref[...]
    st = st_ref[...]
    dot = jnp.dot(qs.astype(jnp.bfloat16), st.astype(jnp.bfloat16),
                  preferred_element_type=jnp.float32)
    d = (q2_ref[...] - 2.0 * dot) + s2_ref[...]

    u = jax.lax.bitcast_convert_type(d, jnp.int32)
    key = jnp.where(u >= 0, u, INT_MIN - u)

    def bs_step(_, lohi):
        lo, hi = lohi
        mid = (lo >> 1) + (hi >> 1) + (lo & hi & 1)
        cnt = jnp.sum((key <= mid).astype(jnp.int32), axis=1, keepdims=True)
        ge = cnt >= (k + 1)
        return jnp.where(ge, lo, mid + 1), jnp.where(ge, mid, hi)

    lo0 = jnp.full((Q, 1), INT_MIN, jnp.int32)
    hi0 = jnp.full((Q, 1), 0x7F7FFFFF, jnp.int32)
    lo, hi = jax.lax.fori_loop(0, 32, bs_step, (lo0, hi0))

    out_ref[...] = jnp.where(key <= hi, d, BIG)


def _gravnet_agg(s, flr, k):
    sp = jnp.concatenate(
        [s, jnp.full((NPAD - N, SPACE_DIM), PAD_COORD, jnp.float32)], axis=0)
    s2 = jnp.sum(sp * sp, axis=1)
    dm = pl.pallas_call(
        functools.partial(_select_kernel_body, k),
        grid=(NPAD // Q,),
        in_specs=[
            pl.BlockSpec((Q, SPACE_DIM), lambda i: (i, 0)),
            pl.BlockSpec((SPACE_DIM, NPAD), lambda i: (0, 0)),
            pl.BlockSpec((Q, 1), lambda i: (i, 0)),
            pl.BlockSpec((1, NPAD), lambda i: (0, 0)),
        ],
        out_specs=pl.BlockSpec((Q, NPAD), lambda i: (i, 0)),
        out_shape=jax.ShapeDtypeStruct((NPAD, NPAD), jnp.float32),
    )(sp, sp.T, s2.reshape(NPAD, 1), s2.reshape(1, NPAD))[:N]

    def f(dchunk):
        negd, idx = jax.lax.top_k(-dchunk, k + 1)
        idx = idx[:, 1:]
        dsq = jnp.clip(-negd[:, 1:], 0.0, None)
        w = jnp.exp(-10.0 * dsq)
        nb = flr[idx] * w[..., None]
        return jnp.concatenate([jnp.mean(nb, axis=1), jnp.max(nb, axis=1)], axis=1)

    out = jax.lax.map(f, dm.reshape(N // C, C, NPAD))
    return out.reshape(N, 2 * PROP_DIM)


def _bn(x, g, b):
    mu = jnp.mean(x, axis=0)
    var = jnp.var(x, axis=0)
    return g * (x - mu) / jnp.sqrt(var + 1e-5) + b


def _block(x, p, i, k):
    xn = _bn(x, p['blk%d_bn_g' % i], p['blk%d_bn_b' % i])
    s = xn @ p['blk%d_Ws' % i].T + p['blk%d_bs' % i]
    flr = xn @ p['blk%d_Wf' % i].T + p['blk%d_bf' % i]
    agg = _gravnet_agg(s, flr, k)
    h = jnp.concatenate([xn, agg], axis=1)
    return jax.nn.elu(h @ p['blk%d_Wo' % i].T + p['blk%d_bo' % i])


def kernel(coord, feat, offset, segment, params):
    p = params
    x = _bn(feat, p['bn1_g'], p['bn1_b'])
    x = x @ p['dense1_W'].T
    allfeat = [x]
    for i, k in enumerate(K_LIST):
        out = _block(x, p, i, k)
        allfeat.append(out)
        x = jnp.concatenate(allfeat, axis=1)
    x = jnp.concatenate(allfeat, axis=-1)
    for i in range(3):
        x = jax.nn.elu(x @ p['post%d_W' % i].T + p['post%d_b' % i])
    x = _bn(x, p['bn2_g'], p['bn2_b'])
    return x @ p['clust_W'].T
